# baseline (speedup 1.0000x reference)
import functools

import jax
import jax.numpy as jnp
from jax import lax
from jax.experimental import pallas as pl
from jax.experimental.pallas import tpu as pltpu
from jax.experimental.pallas import tpu_sc as plsc

VOCAB = 101
OUT = 128
BATCH = 16384
TBL = VOCAB * OUT

NUM_CORES = 2
NUM_SUBCORES = 16
NW = NUM_CORES * NUM_SUBCORES
BPW = BATCH // NW
CH = 128
NCH = BPW // CH
CPT = OUT // NUM_SUBCORES

_MESH = plsc.VectorSubcoreMesh(core_axis_name="c", subcore_axis_name="s")


@functools.partial(
    pl.kernel,
    mesh=_MESH,
    compiler_params=pltpu.CompilerParams(needs_layout_passes=False),
    out_type=(
        jax.ShapeDtypeStruct((BATCH, OUT), jnp.float32),
        jax.ShapeDtypeStruct((NUM_CORES * OUT, OUT), jnp.float32),
    ),
    scratch_types=[
        pltpu.VMEM((16384,), jnp.float32),
        pltpu.VMEM((CPT, OUT), jnp.float32),
        pltpu.VMEM((BPW,), jnp.int32),
        pltpu.VMEM((2, CH, OUT), jnp.float32),
        pltpu.SemaphoreType.DMA,
        pltpu.SemaphoreType.DMA,
        pltpu.SemaphoreType.DMA,
    ],
)
def _lookup(w_hbm, idx_hbm, out_hbm, tbl_hbm, w_v, stage_v, idx_v, buf_v,
            g_sem, w_sem, i_sem):
    c = lax.axis_index("c")
    s = lax.axis_index("s")
    wid = s * NUM_CORES + c
    base = wid * BPW

    cp_w = pltpu.async_copy(w_hbm, w_v.at[pl.ds(0, TBL)], i_sem)
    cp_i = pltpu.async_copy(idx_hbm.at[pl.ds(base, BPW)], idx_v, i_sem)
    cp_w.wait()

    lane = lax.iota(jnp.int32, 16)
    col0 = s * CPT

    def prep_body(i, _):
        gidx = (col0 + i // 8) + VOCAB * ((i % 8) * 16) + VOCAB * lane
        v = plsc.load_gather(w_v, [gidx])
        stage_v[i // 8, pl.ds((i % 8) * 16, 16)] = 1.0 / (1.0 + jnp.exp(-v))
        return 0

    lax.fori_loop(0, CPT * (OUT // 16), prep_body, 0, unroll=4)
    pltpu.sync_copy(stage_v, tbl_hbm.at[pl.ds(c * OUT + col0, CPT)])
    cp_i.wait()

    def rebase_body(i, _):
        idx_v[pl.ds(i * 16, 16)] = idx_v[pl.ds(i * 16, 16)] + c * OUT
        return 0

    lax.fori_loop(0, BPW // 16, rebase_body, 0, unroll=4)
    plsc.subcore_barrier()

    gs = [None] * NCH
    ws = [None] * NCH
    gs[0] = pltpu.async_copy(
        tbl_hbm.at[idx_v.at[pl.ds(0, CH)]], buf_v.at[0], g_sem)
    for k in range(NCH):
        gs[k].wait()
        if k + 1 < NCH:
            if k >= 1:
                ws[k - 1].wait()
            gs[k + 1] = pltpu.async_copy(
                tbl_hbm.at[idx_v.at[pl.ds((k + 1) * CH, CH)]],
                buf_v.at[(k + 1) % 2], g_sem)
        ws[k] = pltpu.async_copy(
            buf_v.at[k % 2], out_hbm.at[pl.ds(base + k * CH, CH)], w_sem)
    ws[NCH - 2].wait()
    ws[NCH - 1].wait()


def kernel(x, W):
    out, _ = _lookup(W.reshape(-1), x)
    return out


# speedup vs baseline: 1.5294x; 1.5294x over previous
import functools

import jax
import jax.numpy as jnp
from jax import lax
from jax.experimental import pallas as pl
from jax.experimental.pallas import tpu as pltpu
from jax.experimental.pallas import tpu_sc as plsc

VOCAB = 101
OUT = 128
BATCH = 16384
TBL = VOCAB * OUT

NUM_CORES = 2
NUM_SUBCORES = 16
NW = NUM_CORES * NUM_SUBCORES
BPW = BATCH // NW
CH = 64
NCH = BPW // CH
CPT = OUT // NUM_SUBCORES

_MESH = plsc.VectorSubcoreMesh(core_axis_name="c", subcore_axis_name="s")


@functools.partial(
    pl.kernel,
    mesh=_MESH,
    compiler_params=pltpu.CompilerParams(needs_layout_passes=False),
    out_type=jax.ShapeDtypeStruct((BATCH, OUT), jnp.float32),
    scratch_types=[
        pltpu.VMEM((16384,), jnp.float32),
        pltpu.VMEM((CPT, OUT), jnp.float32),
        pltpu.VMEM((BPW,), jnp.int32),
        pltpu.VMEM((2, CH, OUT), jnp.float32),
        pltpu.VMEM_SHARED((OUT, OUT), jnp.float32),
        pltpu.SemaphoreType.DMA,
        pltpu.SemaphoreType.DMA,
        pltpu.SemaphoreType.DMA,
    ],
)
def _lookup(w_hbm, idx_hbm, out_hbm, w_v, stage_v, idx_v, buf_v, tbl_sh,
            g_sem, w_sem, i_sem):
    c = lax.axis_index("c")
    s = lax.axis_index("s")
    wid = s * NUM_CORES + c
    base = wid * BPW

    cp_w = pltpu.async_copy(w_hbm, w_v.at[pl.ds(0, TBL)], i_sem)
    cp_i = pltpu.async_copy(idx_hbm.at[pl.ds(base, BPW)], idx_v, i_sem)
    cp_w.wait()

    lane = lax.iota(jnp.int32, 16)
    col0 = s * CPT

    def prep_body(i, _):
        vals = []
        for u in range(4):
            ii = i * 4 + u
            gidx = (col0 + ii // 8) + VOCAB * ((ii % 8) * 16) + VOCAB * lane
            vals.append(plsc.load_gather(w_v, [gidx]))
        for u in range(4):
            ii = i * 4 + u
            stage_v[ii // 8, pl.ds((ii % 8) * 16, 16)] = (
                1.0 / (1.0 + jnp.exp(-vals[u])))
        return 0

    lax.fori_loop(0, CPT * (OUT // 16) // 4, prep_body, 0)
    pltpu.sync_copy(stage_v, tbl_sh.at[pl.ds(col0, CPT)])
    plsc.subcore_barrier()
    cp_i.wait()

    gs = [None] * NCH
    ws = [None] * NCH
    gs[0] = pltpu.async_copy(
        tbl_sh.at[idx_v.at[pl.ds(0, CH)]], buf_v.at[0], g_sem)
    for k in range(NCH):
        gs[k].wait()
        if k + 1 < NCH:
            if k >= 1:
                ws[k - 1].wait()
            gs[k + 1] = pltpu.async_copy(
                tbl_sh.at[idx_v.at[pl.ds((k + 1) * CH, CH)]],
                buf_v.at[(k + 1) % 2], g_sem)
        ws[k] = pltpu.async_copy(
            buf_v.at[k % 2], out_hbm.at[pl.ds(base + k * CH, CH)], w_sem)
    ws[NCH - 2].wait()
    ws[NCH - 1].wait()


def kernel(x, W):
    return _lookup(W.reshape(-1), x)


# speedup vs baseline: 1.5376x; 1.0053x over previous
import functools

import jax
import jax.numpy as jnp
from jax import lax
from jax.experimental import pallas as pl
from jax.experimental.pallas import tpu as pltpu
from jax.experimental.pallas import tpu_sc as plsc

VOCAB = 101
OUT = 128
BATCH = 16384
TBL = VOCAB * OUT

NUM_CORES = 2
NUM_SUBCORES = 16
NW = NUM_CORES * NUM_SUBCORES
BPW = BATCH // NW
CH = 64
NCH = BPW // CH
CPT = OUT // NUM_SUBCORES

_MESH = plsc.VectorSubcoreMesh(core_axis_name="c", subcore_axis_name="s")


@functools.partial(
    pl.kernel,
    mesh=_MESH,
    compiler_params=pltpu.CompilerParams(needs_layout_passes=False),
    out_type=jax.ShapeDtypeStruct((BATCH, OUT), jnp.float32),
    scratch_types=[
        pltpu.VMEM((16384,), jnp.float32),
        pltpu.VMEM((CPT, OUT), jnp.float32),
        pltpu.VMEM((BPW,), jnp.int32),
        pltpu.VMEM((2, CH, OUT), jnp.float32),
        pltpu.VMEM_SHARED((OUT, OUT), jnp.float32),
        pltpu.SemaphoreType.DMA,
        pltpu.SemaphoreType.DMA,
        pltpu.SemaphoreType.DMA,
    ],
)
def _lookup(w_hbm, idx_hbm, out_hbm, w_v, stage_v, idx_v, buf_v, tbl_sh,
            g_sem, w_sem, i_sem):
    c = lax.axis_index("c")
    s = lax.axis_index("s")
    wid = s * NUM_CORES + c
    base = wid * BPW

    cp_w = pltpu.async_copy(w_hbm, w_v.at[pl.ds(0, TBL)], i_sem)
    cp_i = pltpu.async_copy(idx_hbm.at[pl.ds(base, BPW)], idx_v, i_sem)
    cp_w.wait()

    lane = lax.iota(jnp.int32, 16)
    col0 = s * CPT

    def prep_body(i, _):
        vals = []
        for u in range(8):
            ii = i * 8 + u
            gidx = (col0 + ii // 8) + VOCAB * ((ii % 8) * 16) + VOCAB * lane
            vals.append(plsc.load_gather(w_v, [gidx]))
        for u in range(8):
            ii = i * 8 + u
            stage_v[ii // 8, pl.ds((ii % 8) * 16, 16)] = (
                1.0 / (1.0 + jnp.exp(-vals[u])))
        return 0

    lax.fori_loop(0, CPT * (OUT // 16) // 8, prep_body, 0)
    pltpu.sync_copy(stage_v, tbl_sh.at[pl.ds(col0, CPT)])
    plsc.subcore_barrier()
    cp_i.wait()

    gs = [None] * NCH
    ws = [None] * NCH
    gs[0] = pltpu.async_copy(
        tbl_sh.at[idx_v.at[pl.ds(0, CH)]], buf_v.at[0], g_sem)
    for k in range(NCH):
        gs[k].wait()
        if k + 1 < NCH:
            if k >= 1:
                ws[k - 1].wait()
            gs[k + 1] = pltpu.async_copy(
                tbl_sh.at[idx_v.at[pl.ds((k + 1) * CH, CH)]],
                buf_v.at[(k + 1) % 2], g_sem)
        ws[k] = pltpu.async_copy(
            buf_v.at[k % 2], out_hbm.at[pl.ds(base + k * CH, CH)], w_sem)
    ws[NCH - 2].wait()
    ws[NCH - 1].wait()


def kernel(x, W):
    return _lookup(W.reshape(-1), x)


# speedup vs baseline: 1.5399x; 1.0015x over previous
import functools

import jax
import jax.numpy as jnp
from jax import lax
from jax.experimental import pallas as pl
from jax.experimental.pallas import tpu as pltpu
from jax.experimental.pallas import tpu_sc as plsc

VOCAB = 101
OUT = 128
BATCH = 16384
TBL = VOCAB * OUT

NUM_CORES = 2
NUM_SUBCORES = 16
NW = NUM_CORES * NUM_SUBCORES
BPW = BATCH // NW
CH = 64
NCH = BPW // CH
CPT = OUT // NUM_SUBCORES

_MESH = plsc.VectorSubcoreMesh(core_axis_name="c", subcore_axis_name="s")


@functools.partial(
    pl.kernel,
    mesh=_MESH,
    compiler_params=pltpu.CompilerParams(needs_layout_passes=False),
    out_type=jax.ShapeDtypeStruct((BATCH, OUT), jnp.float32),
    scratch_types=[
        pltpu.VMEM((16384,), jnp.float32),
        pltpu.VMEM((CPT, OUT), jnp.float32),
        pltpu.VMEM((BPW,), jnp.int32),
        pltpu.VMEM((3, CH, OUT), jnp.float32),
        pltpu.VMEM_SHARED((OUT, OUT), jnp.float32),
        pltpu.SemaphoreType.DMA,
        pltpu.SemaphoreType.DMA,
        pltpu.SemaphoreType.DMA,
    ],
)
def _lookup(w_hbm, idx_hbm, out_hbm, w_v, stage_v, idx_v, buf_v, tbl_sh,
            g_sem, w_sem, i_sem):
    c = lax.axis_index("c")
    s = lax.axis_index("s")
    wid = s * NUM_CORES + c
    base = wid * BPW

    cp_w = pltpu.async_copy(w_hbm, w_v.at[pl.ds(0, TBL)], i_sem)
    cp_i = pltpu.async_copy(idx_hbm.at[pl.ds(base, BPW)], idx_v, i_sem)
    cp_w.wait()

    lane = lax.iota(jnp.int32, 16)
    col0 = s * CPT

    def prep_body(i, _):
        vals = []
        for u in range(8):
            ii = i * 8 + u
            gidx = (col0 + ii // 8) + VOCAB * ((ii % 8) * 16) + VOCAB * lane
            vals.append(plsc.load_gather(w_v, [gidx]))
        for u in range(8):
            ii = i * 8 + u
            stage_v[ii // 8, pl.ds((ii % 8) * 16, 16)] = (
                1.0 / (1.0 + jnp.exp(-vals[u])))
        return 0

    lax.fori_loop(0, CPT * (OUT // 16) // 8, prep_body, 0)
    pltpu.sync_copy(stage_v, tbl_sh.at[pl.ds(col0, CPT)])
    plsc.subcore_barrier()
    cp_i.wait()

    gs = [None] * NCH
    ws = [None] * NCH
    gs[0] = pltpu.async_copy(
        tbl_sh.at[idx_v.at[pl.ds(0, CH)]], buf_v.at[0], g_sem)
    for k in range(NCH):
        gs[k].wait()
        if k + 1 < NCH:
            if k >= 2:
                ws[k - 2].wait()
            gs[k + 1] = pltpu.async_copy(
                tbl_sh.at[idx_v.at[pl.ds((k + 1) * CH, CH)]],
                buf_v.at[(k + 1) % 3], g_sem)
        ws[k] = pltpu.async_copy(
            buf_v.at[k % 3], out_hbm.at[pl.ds(base + k * CH, CH)], w_sem)
    ws[NCH - 3].wait()
    ws[NCH - 2].wait()
    ws[NCH - 1].wait()


def kernel(x, W):
    return _lookup(W.reshape(-1), x)
